# Initial kernel scaffold; baseline (speedup 1.0000x reference)
import jax
import jax.numpy as jnp
from jax.experimental import pallas as pl


def kernel(x, token_weight, time_weight):
    raise NotImplementedError("write your pallas kernel here")


# speedup vs baseline: 1.1153x; 1.1153x over previous
import jax
import jax.numpy as jnp
from jax import lax
from jax.experimental import pallas as pl
from jax.experimental.pallas import tpu as pltpu


def _rows_kernel(idx_ref, tok_ref, time_ref, o_ref):
    r, k = idx_ref.shape
    kv, kd = tok_ref.shape
    v = kv // k
    trows = time_ref.shape[0]

    col = lax.broadcasted_iota(jnp.int32, (r, kv), 1)
    hot = idx_ref[:, 0:1] == col
    for j in range(1, k):
        hot = jnp.logical_or(hot, (idx_ref[:, j:j + 1] + j * v) == col)
    tok = jnp.dot(hot.astype(jnp.bfloat16), tok_ref[...],
                  preferred_element_type=jnp.float32)
    tpos = time_ref[...]
    o_ref[...] = (tok.reshape(r // trows, trows, kd) + tpos[None]
                  ).reshape(r, kd)


@jax.jit
def kernel(x, token_weight, time_weight):
    b, t = x.shape
    v, d = token_weight.shape
    time_td = jnp.transpose(time_weight[:, :t])

    k = 128 // d if (d < 128 and 128 % d == 0 and t % (128 // d) == 0) else 1
    tp = t // k
    kd = k * d
    n = b * tp

    if k > 1:
        tok_bd = jnp.kron(jnp.eye(k, dtype=token_weight.dtype), token_weight)
    else:
        tok_bd = token_weight
    tok_bd = tok_bd.astype(jnp.bfloat16)
    time_packed = time_td.reshape(tp, kd)
    idx = x.astype(jnp.int32).reshape(n, k)

    rows = 2048
    while n % rows or (rows % tp and tp % rows):
        rows //= 2
    if rows >= tp:
        time_spec = pl.BlockSpec((tp, kd), lambda i: (0, 0),
                                 pipeline_mode=pl.Buffered(1))
    else:
        time_spec = pl.BlockSpec((rows, kd),
                                 lambda i, _m=tp // rows: (i % _m, 0))

    out = pl.pallas_call(
        _rows_kernel,
        out_shape=jax.ShapeDtypeStruct((n, kd), token_weight.dtype),
        grid_spec=pltpu.PrefetchScalarGridSpec(
            num_scalar_prefetch=0,
            grid=(n // rows,),
            in_specs=[
                pl.BlockSpec((rows, k), lambda i: (i, 0)),
                pl.BlockSpec((k * v, kd), lambda i: (0, 0),
                             pipeline_mode=pl.Buffered(1)),
                time_spec,
            ],
            out_specs=pl.BlockSpec((rows, kd), lambda i: (i, 0)),
        ),
        compiler_params=pltpu.CompilerParams(
            dimension_semantics=("parallel",)),
    )(idx, tok_bd, time_packed)
    return out.reshape(b, t, d)


# speedup vs baseline: 2.4885x; 2.2312x over previous
import jax
import jax.numpy as jnp
from jax import lax
from jax.experimental import pallas as pl
from jax.experimental.pallas import tpu as pltpu


def _embed_kernel(x_ref, tokT_ref, time_ref, o_ref):
    bb, t = x_ref.shape
    d, v = tokT_ref.shape

    row = lax.broadcasted_iota(jnp.int32, (v, t), 0)
    tokT = tokT_ref[...]
    timeT = time_ref[...]
    for bi in range(bb):
        xb = jnp.broadcast_to(x_ref[bi:bi + 1, :], (v, t))
        hotT = (xb == row).astype(jnp.bfloat16)
        outT = jnp.dot(tokT, hotT,
                       preferred_element_type=jnp.float32)
        o_ref[bi] = jnp.transpose(outT + timeT)


@jax.jit
def kernel(x, token_weight, time_weight):
    b, t = x.shape
    v, d = token_weight.shape
    tokT = jnp.transpose(token_weight).astype(jnp.bfloat16)
    timeT = time_weight[:, :t]

    bb = 8
    while b % bb:
        bb //= 2

    out = pl.pallas_call(
        _embed_kernel,
        out_shape=jax.ShapeDtypeStruct((b, t, d), token_weight.dtype),
        grid_spec=pltpu.PrefetchScalarGridSpec(
            num_scalar_prefetch=0,
            grid=(b // bb,),
            in_specs=[
                pl.BlockSpec((bb, t), lambda i: (i, 0)),
                pl.BlockSpec((d, v), lambda i: (0, 0),
                             pipeline_mode=pl.Buffered(1)),
                pl.BlockSpec((d, t), lambda i: (0, 0),
                             pipeline_mode=pl.Buffered(1)),
            ],
            out_specs=pl.BlockSpec((bb, t, d), lambda i: (i, 0, 0)),
        ),
        compiler_params=pltpu.CompilerParams(
            dimension_semantics=("parallel",)),
    )(x.astype(jnp.int32), tokT, timeT)
    return out
